# baseline (speedup 1.0000x reference)
import jax
import jax.numpy as jnp
from jax.experimental import pallas as pl
from jax.experimental.pallas import tpu as pltpu


def _conv3x3_kernel(x_ref, w_ref, b_ref, o_ref, xsh_ref):
    c_in = x_ref.shape[1]
    _, hp, w, _ = xsh_ref.shape
    h = hp - 2
    zrow = jnp.zeros((1, w, c_in), xsh_ref.dtype)
    zcol = jnp.zeros((h, 1, c_in), xsh_ref.dtype)

    for img in range(x_ref.shape[0]):
      xt = x_ref[img].astype(jnp.bfloat16).T.reshape(h, w, c_in)

      for kx in range(3):
          xsh_ref[kx, 0:1] = zrow
          xsh_ref[kx, hp - 1:hp] = zrow
      xsh_ref[1, 1:1 + h] = xt
      xsh_ref[0, 1:1 + h, 0:1] = zcol
      xsh_ref[0, 1:1 + h, 1:w] = xt[:, 0:w - 1]
      xsh_ref[2, 1:1 + h, 0:w - 1] = xt[:, 1:w]
      xsh_ref[2, 1:1 + h, w - 1:w] = zcol

      acc = b_ref[...]
      for ky in range(3):
          for kx in range(3):
              t = ky * 3 + kx
              acc = acc + jax.lax.dot_general(
                  w_ref[t * c_in:(t + 1) * c_in, :],
                  xsh_ref[kx, ky:ky + h].reshape(h * w, c_in),
                  dimension_numbers=(((0,), (1,)), ((), ())),
                  preferred_element_type=jnp.float32,
              )
      o_ref[img] = acc


def kernel(x, weight_norm, bias, scale):
    b, c_in, h, w = x.shape
    c_out, _, k_size, _ = weight_norm.shape
    k_dim = k_size * k_size * c_in

    w_mat = (weight_norm * jnp.asarray(scale, weight_norm.dtype)
             ).transpose(2, 3, 1, 0).reshape(k_dim, c_out).astype(jnp.bfloat16)

    x3 = x.reshape(b, c_in, h * w)
    bias_col = bias.astype(jnp.float32).reshape(c_out, 1)

    out = pl.pallas_call(
        _conv3x3_kernel,
        out_shape=jax.ShapeDtypeStruct((b, c_out, h * w), jnp.float32),
        grid=(b // 2,),
        in_specs=[
            pl.BlockSpec((2, c_in, h * w), lambda i: (i, 0, 0)),
            pl.BlockSpec((k_dim, c_out), lambda i: (0, 0)),
            pl.BlockSpec((c_out, 1), lambda i: (0, 0)),
        ],
        out_specs=pl.BlockSpec((2, c_out, h * w), lambda i: (i, 0, 0)),
        scratch_shapes=[
            pltpu.VMEM((3, h + 2, w, c_in), jnp.bfloat16),
        ],
        compiler_params=pltpu.CompilerParams(
            dimension_semantics=("parallel",),
            vmem_limit_bytes=40 * 1024 * 1024,
        ),
    )(x3, w_mat, bias_col)

    return out.reshape(b, c_out, h, w).astype(x.dtype)


# speedup vs baseline: 1.0286x; 1.0286x over previous
import jax
import jax.numpy as jnp
from jax.experimental import pallas as pl
from jax.experimental.pallas import tpu as pltpu


def _conv3x3_kernel(x_ref, w_ref, b_ref, o_ref, xsh_ref):
    c_in = x_ref.shape[1]
    _, hp, w, _ = xsh_ref.shape
    h = hp - 2
    zrow = jnp.zeros((1, w, c_in), xsh_ref.dtype)
    zcol = jnp.zeros((h, 1, c_in), xsh_ref.dtype)

    xt = x_ref[0].astype(jnp.bfloat16).T.reshape(h, w, c_in)

    for kx in range(3):
        xsh_ref[kx, 0:1] = zrow
        xsh_ref[kx, hp - 1:hp] = zrow
    xsh_ref[1, 1:1 + h] = xt
    xsh_ref[0, 1:1 + h, 0:1] = zcol
    xsh_ref[0, 1:1 + h, 1:w] = xt[:, 0:w - 1]
    xsh_ref[2, 1:1 + h, 0:w - 1] = xt[:, 1:w]
    xsh_ref[2, 1:1 + h, w - 1:w] = zcol

    acc = b_ref[...]
    for ky in range(3):
        for kx in range(3):
            t = ky * 3 + kx
            acc = acc + jax.lax.dot_general(
                w_ref[t * c_in:(t + 1) * c_in, :],
                xsh_ref[kx, ky:ky + h].reshape(h * w, c_in),
                dimension_numbers=(((0,), (1,)), ((), ())),
                preferred_element_type=jnp.float32,
            )
    o_ref[0] = acc


def kernel(x, weight_norm, bias, scale):
    b, c_in, h, w = x.shape
    c_out, _, k_size, _ = weight_norm.shape
    k_dim = k_size * k_size * c_in

    w_mat = (weight_norm * jnp.asarray(scale, weight_norm.dtype)
             ).transpose(2, 3, 1, 0).reshape(k_dim, c_out).astype(jnp.bfloat16)

    x3 = x.reshape(b, c_in, h * w)
    bias_col = bias.astype(jnp.float32).reshape(c_out, 1)

    out = pl.pallas_call(
        _conv3x3_kernel,
        out_shape=jax.ShapeDtypeStruct((b, c_out, h * w), jnp.float32),
        grid=(b,),
        in_specs=[
            pl.BlockSpec((1, c_in, h * w), lambda i: (i, 0, 0)),
            pl.BlockSpec((k_dim, c_out), lambda i: (0, 0)),
            pl.BlockSpec((c_out, 1), lambda i: (0, 0)),
        ],
        out_specs=pl.BlockSpec((1, c_out, h * w), lambda i: (i, 0, 0)),
        scratch_shapes=[
            pltpu.VMEM((3, h + 2, w, c_in), jnp.bfloat16),
        ],
        compiler_params=pltpu.CompilerParams(
            dimension_semantics=("parallel",),
            vmem_limit_bytes=40 * 1024 * 1024,
        ),
    )(x3, w_mat, bias_col)

    return out.reshape(b, c_out, h, w).astype(x.dtype)
